# baseline (speedup 1.0000x reference)
import functools
import math

import jax
import jax.numpy as jnp
from jax import lax
from jax.experimental import pallas as pl
from jax.experimental.pallas import tpu as pltpu
from jax.experimental.pallas import tpu_sc as plsc

_N = 10000
_E = 160000
_F = 16
_OUT = 8

_NC = 2
_NS = 16
_NW = _NC * _NS
_EPW = _E // _NW
_EPT = _E // _NS
_CH = 625
_GCH = _EPW // _CH
_SCH = _EPT // _CH
_IDXROWS = _E // _CH

_NP = 10240
_RPT = _NP // _NS

_STG = 1250
_SRND = _EPT // _STG
_GRND = _EPW // _STG
_CPR = _STG // _CH

_EROWS = _E * _F // 128
_NROWS = _NP * _F // 128
_GRID = 20
_BER = _EROWS // _GRID
_NBR = _NROWS // _GRID

_SC_PARAMS = pltpu.CompilerParams(use_tc_tiling_on_sc=False)
_MESH_CACHE = []


def _mesh():
    if not _MESH_CACHE:
        _MESH_CACHE.append(plsc.VectorSubcoreMesh(
            core_axis_name="c", subcore_axis_name="s",
            num_cores=_NC, num_subcores=_NS))
    return _MESH_CACHE[0]


def _sc_gather_count(x, src2d, dst2d, ones_blk, zeros_blk):
    @functools.partial(
        pl.kernel,
        out_type=[
            jax.ShapeDtypeStruct((_E, _F), jnp.float32),
            jax.ShapeDtypeStruct((_NP, _F), jnp.float32),
        ],
        mesh=_mesh(),
        compiler_params=_SC_PARAMS,
        scratch_types=[
            pltpu.VMEM((_GCH, _CH), jnp.int32),
            pltpu.VMEM((_SCH, _CH), jnp.int32),
            pltpu.VMEM((_STG, _F), jnp.float32),
            pltpu.VMEM((_CH, _F), jnp.float32),
            pltpu.SemaphoreType.DMA,
            pltpu.SemaphoreType.DMA,
            pltpu.VMEM_SHARED((_NP, _F), jnp.float32),
        ],
    )
    def k(x_hbm, src_hbm, dst_hbm, ones_hbm, zeros_hbm, xs_out, cnt_out,
          src_v, dst_v, stg_v, ones_v, gsem, ssem, acc):
        c = lax.axis_index("c")
        s = lax.axis_index("s")
        wid = c * _NS + s
        base = wid * _EPW
        pltpu.sync_copy(src_hbm.at[pl.ds(wid * _GCH, _GCH)], src_v)
        pltpu.sync_copy(dst_hbm.at[pl.ds(s * _SCH, _SCH)], dst_v)
        pltpu.sync_copy(ones_hbm, ones_v)
        pltpu.sync_copy(zeros_hbm, acc.at[pl.ds(s * _RPT, _RPT)])
        plsc.subcore_barrier()
        sds = []
        for j in range(_SCH):
            sds.append(pltpu.async_copy(
                ones_v, acc.at[dst_v.at[j]], ssem, add=True))
        for r in range(_GRND):
            gds = []
            for j in range(_CPR):
                gds.append(pltpu.async_copy(
                    x_hbm.at[src_v.at[r * _CPR + j]],
                    stg_v.at[pl.ds(j * _CH, _CH)], gsem))
            for d in gds:
                d.wait()
            pltpu.sync_copy(stg_v, xs_out.at[pl.ds(base + r * _STG, _STG)])
        for d in sds:
            d.wait()
        plsc.subcore_barrier()
        @pl.when(c == 0)
        def _():
            pltpu.sync_copy(acc.at[pl.ds(s * _RPT, _RPT)],
                            cnt_out.at[pl.ds(s * _RPT, _RPT)])

    return k(x, src2d, dst2d, ones_blk, zeros_blk)


def _sc_scatter_finish_gather(msg, dst2d, src2d, cnt, xroot, gvec, bvec,
                              zeros_blk, write_node_out):
    out_type = [jax.ShapeDtypeStruct((_E, _F), jnp.float32)]
    if write_node_out:
        out_type.append(jax.ShapeDtypeStruct((_NP, _F), jnp.float32))

    @functools.partial(
        pl.kernel,
        out_type=out_type,
        mesh=_mesh(),
        compiler_params=_SC_PARAMS,
        scratch_types=[
            pltpu.VMEM((_SCH, _CH), jnp.int32),
            pltpu.VMEM((_GCH, _CH), jnp.int32),
            pltpu.VMEM((_STG, _F), jnp.float32),
            pltpu.VMEM((_RPT, _F), jnp.float32),
            pltpu.VMEM((_RPT, _F), jnp.float32),
            pltpu.VMEM((_RPT, _F), jnp.float32),
            pltpu.VMEM((_F,), jnp.float32),
            pltpu.VMEM((_F,), jnp.float32),
            pltpu.SemaphoreType.DMA,
            pltpu.SemaphoreType.DMA,
            pltpu.VMEM_SHARED((_NP, _F), jnp.float32),
        ],
    )
    def k(msg_hbm, dst_hbm, src_hbm, cnt_hbm, xr_hbm, g_hbm, b_hbm,
          zeros_hbm, hs_out, *rest):
        if write_node_out:
            h_out = rest[0]
            (dst_v, src_v, stg_v, cnt_v, xr_v, acc_v,
             g_v, b_v, ssem, gsem, acc) = rest[1:]
        else:
            (dst_v, src_v, stg_v, cnt_v, xr_v, acc_v,
             g_v, b_v, ssem, gsem, acc) = rest
        c = lax.axis_index("c")
        s = lax.axis_index("s")
        wid = c * _NS + s
        base = wid * _EPW
        pltpu.sync_copy(dst_hbm.at[pl.ds(s * _SCH, _SCH)], dst_v)
        pltpu.sync_copy(src_hbm.at[pl.ds(wid * _GCH, _GCH)], src_v)
        pltpu.sync_copy(cnt_hbm.at[pl.ds(s * _RPT, _RPT)], cnt_v)
        pltpu.sync_copy(xr_hbm.at[pl.ds(s * _RPT, _RPT)], xr_v)
        pltpu.sync_copy(g_hbm, g_v)
        pltpu.sync_copy(b_hbm, b_v)
        pltpu.sync_copy(zeros_hbm, acc.at[pl.ds(s * _RPT, _RPT)])
        plsc.subcore_barrier()
        for r in range(_SRND):
            pltpu.sync_copy(
                msg_hbm.at[pl.ds(s * _EPT + r * _STG, _STG)], stg_v)
            sds = []
            for j in range(_CPR):
                sds.append(pltpu.async_copy(
                    stg_v.at[pl.ds(j * _CH, _CH)],
                    acc.at[dst_v.at[r * _CPR + j]], ssem, add=True))
            for d in sds:
                d.wait()
        plsc.subcore_barrier()
        pltpu.sync_copy(acc.at[pl.ds(s * _RPT, _RPT)], acc_v)
        gv = g_v[...]
        bv = b_v[...]

        def row(i, _):
            agg = acc_v[i] / jnp.maximum(cnt_v[i], 1.0)
            acc_v[i] = jnp.maximum((agg + xr_v[i]) * gv + bv, 0.0)
            return 0

        lax.fori_loop(0, _RPT, row, 0)
        pltpu.sync_copy(acc_v, acc.at[pl.ds(s * _RPT, _RPT)])
        if write_node_out:
            @pl.when(c == 0)
            def _():
                pltpu.sync_copy(acc_v, h_out.at[pl.ds(s * _RPT, _RPT)])
        plsc.subcore_barrier()
        for r in range(_GRND):
            gds = []
            for j in range(_CPR):
                gds.append(pltpu.async_copy(
                    acc.at[src_v.at[r * _CPR + j]],
                    stg_v.at[pl.ds(j * _CH, _CH)], gsem))
            for d in gds:
                d.wait()
            pltpu.sync_copy(stg_v, hs_out.at[pl.ds(base + r * _STG, _STG)])

    return k(msg, dst2d, src2d, cnt, xroot, gvec, bvec, zeros_blk)


def _msg_consts(w2, b2):
    f = _F
    k_idx = jnp.tile(jnp.arange(f), (f,))
    o_idx = jnp.repeat(jnp.arange(f), f)
    s_mat = (jnp.arange(f)[:, None] == k_idx[None, :]).astype(jnp.float32)
    c_mat = (o_idx[:, None] == jnp.arange(f)[None, :]).astype(jnp.float32)
    u_mat = w2.reshape(f, f, f).transpose(0, 2, 1)[:, k_idx, o_idx]
    b2m = b2.reshape(f, f)
    eye8 = jnp.eye(8, dtype=jnp.float32)
    return (jnp.kron(eye8, s_mat), jnp.kron(eye8, u_mat),
            jnp.kron(eye8, c_mat), jnp.kron(eye8, b2m))


def _bd8(w):
    return jnp.kron(jnp.eye(8, dtype=jnp.float32), w)


def _tile8(b):
    return jnp.tile(b.reshape(1, -1), (1, 8))


def _tc_msg(ea_pk, xs_pk, xn_pk, w1t8, b1t, s8, u8, c8, b8, root8, biast):
    def body(ea_ref, xs_ref, xn_ref, w1t_ref, b1_ref, s_ref, u_ref, c_ref,
             b8_ref, root_ref, biast_ref, out_ref, xr_ref):
        he = jnp.maximum(
            jnp.dot(ea_ref[...], w1t_ref[...],
                    preferred_element_type=jnp.float32) + b1_ref[...], 0.0)
        xsb = xs_ref[...]
        acc = jnp.dot(xsb, b8_ref[...], preferred_element_type=jnp.float32)
        for j in range(8):
            thj = jnp.dot(he, s_ref[:, j * 256:(j + 1) * 256],
                          preferred_element_type=jnp.float32)
            gj = jnp.dot(xsb, u_ref[:, j * 256:(j + 1) * 256],
                         preferred_element_type=jnp.float32)
            acc = acc + jnp.dot(thj * gj, c_ref[j * 256:(j + 1) * 256, :],
                                preferred_element_type=jnp.float32)
        out_ref[...] = acc
        xr_ref[...] = jnp.dot(
            xn_ref[...], root_ref[...],
            preferred_element_type=jnp.float32) + biast_ref[...]

    l = 128
    ll = 2048
    return pl.pallas_call(
        body,
        grid=(_GRID,),
        in_specs=[
            pl.BlockSpec((_BER, l), lambda i: (i, 0)),
            pl.BlockSpec((_BER, l), lambda i: (i, 0)),
            pl.BlockSpec((_NBR, l), lambda i: (i, 0)),
            pl.BlockSpec((l, l), lambda i: (0, 0)),
            pl.BlockSpec((1, l), lambda i: (0, 0)),
            pl.BlockSpec((l, ll), lambda i: (0, 0)),
            pl.BlockSpec((l, ll), lambda i: (0, 0)),
            pl.BlockSpec((ll, l), lambda i: (0, 0)),
            pl.BlockSpec((l, l), lambda i: (0, 0)),
            pl.BlockSpec((l, l), lambda i: (0, 0)),
            pl.BlockSpec((1, l), lambda i: (0, 0)),
        ],
        out_specs=[
            pl.BlockSpec((_BER, l), lambda i: (i, 0)),
            pl.BlockSpec((_NBR, l), lambda i: (i, 0)),
        ],
        out_shape=[
            jax.ShapeDtypeStruct((_EROWS, l), jnp.float32),
            jax.ShapeDtypeStruct((_NROWS, l), jnp.float32),
        ],
    )(ea_pk, xs_pk, xn_pk, w1t8, b1t, s8, u8, c8, b8, root8, biast)


def _tc_edge_mlp(hs2_pk2, l1t16, l1bt, l2t16, l2bt):
    def body(h_ref, l1t_ref, l1b_ref, l2t_ref, l2b_ref, out_ref):
        h3 = jnp.maximum(
            jnp.dot(h_ref[...], l1t_ref[...],
                    preferred_element_type=jnp.float32) + l1b_ref[...], 0.0)
        out_ref[...] = jnp.dot(
            h3, l2t_ref[...], preferred_element_type=jnp.float32) + l2b_ref[...]

    rows = _E // 16
    blk = rows // 10
    return pl.pallas_call(
        body,
        grid=(10,),
        in_specs=[
            pl.BlockSpec((blk, 256), lambda i: (i, 0)),
            pl.BlockSpec((256, 128), lambda i: (0, 0)),
            pl.BlockSpec((1, 128), lambda i: (0, 0)),
            pl.BlockSpec((128, 128), lambda i: (0, 0)),
            pl.BlockSpec((1, 128), lambda i: (0, 0)),
        ],
        out_specs=pl.BlockSpec((blk, 128), lambda i: (i, 0)),
        out_shape=jax.ShapeDtypeStruct((rows, 128), jnp.float32),
    )(hs2_pk2, l1t16, l1bt, l2t16, l2bt)


def _bd16(w):
    return jnp.kron(jnp.eye(16, dtype=jnp.float32), w)


def _tile16(b):
    return jnp.tile(b.reshape(1, -1), (1, 16))


def kernel(x, edge_index, edge_attr, nn1_w1, nn1_b1, nn1_w2, nn1_b2, root1,
           bias1, bn1_g, bn1_b, nn2_w1, nn2_b1, nn2_w2, nn2_b2, root2, bias2,
           bn2_g, bn2_b, lin1_w, lin1_b, lin2_w, lin2_b):
    src2d = edge_index[0].reshape(_IDXROWS, _CH)
    dst2d = edge_index[1].reshape(_IDXROWS, _CH)
    ones_blk = jnp.ones((_CH, _F), jnp.float32)
    zeros_blk = jnp.zeros((_RPT, _F), jnp.float32)

    bn_scale = 1.0 / math.sqrt(1.0 + 1e-5)
    g1 = bn1_g * bn_scale
    g2 = bn2_g * bn_scale

    s1, u1, c1, b1m = _msg_consts(nn1_w2, nn1_b2)
    s2, u2, c2, b2m = _msg_consts(nn2_w2, nn2_b2)

    xp = jnp.pad(x, ((0, _NP - _N), (0, 0)))
    ea_pk = edge_attr.reshape(_EROWS, 128)

    xs, cnt = _sc_gather_count(xp, src2d, dst2d, ones_blk, zeros_blk)
    msg1_pk, xr1_pk = _tc_msg(ea_pk, xs.reshape(_EROWS, 128),
                              xp.reshape(_NROWS, 128),
                              _bd8(nn1_w1.T), _tile8(nn1_b1), s1, u1, c1, b1m,
                              _bd8(root1), _tile8(bias1))
    hs, h1 = _sc_scatter_finish_gather(
        msg1_pk.reshape(_E, _F), dst2d, src2d, cnt,
        xr1_pk.reshape(_NP, _F), g1, bn1_b, zeros_blk, True)
    msg2_pk, xr2_pk = _tc_msg(ea_pk, hs.reshape(_EROWS, 128),
                              h1.reshape(_NROWS, 128),
                              _bd8(nn2_w1.T), _tile8(nn2_b1), s2, u2, c2, b2m,
                              _bd8(root2), _tile8(bias2))
    hs2 = _sc_scatter_finish_gather(
        msg2_pk.reshape(_E, _F), dst2d, src2d, cnt,
        xr2_pk.reshape(_NP, _F), g2, bn2_b, zeros_blk, False)[0]
    out_pk = _tc_edge_mlp(hs2.reshape(_E // 16, 256),
                          _bd16(lin1_w.T), _tile16(lin1_b),
                          _bd16(lin2_w.T), _tile16(lin2_b))
    return out_pk.reshape(_E, _OUT)


# speedup vs baseline: 1.0146x; 1.0146x over previous
import functools
import math

import jax
import jax.numpy as jnp
from jax import lax
from jax.experimental import pallas as pl
from jax.experimental.pallas import tpu as pltpu
from jax.experimental.pallas import tpu_sc as plsc

_N = 10000
_E = 160000
_F = 16
_OUT = 8

_NC = 2
_NS = 16
_NW = _NC * _NS
_EPW = _E // _NW
_EPT = _E // _NS
_CH = 625
_GCH = _EPW // _CH
_SCH = _EPT // _CH
_IDXROWS = _E // _CH

_NP = 10240
_RPT = _NP // _NS

_STG = 1250
_SRND = _EPT // _STG
_GRND = _EPW // _STG
_CPR = _STG // _CH

_EROWS = _E * _F // 128
_NROWS = _NP * _F // 128
_GRID = 20
_BER = _EROWS // _GRID
_NBR = _NROWS // _GRID

_SC_PARAMS = pltpu.CompilerParams(use_tc_tiling_on_sc=False)
_MESH_CACHE = []


def _mesh():
    if not _MESH_CACHE:
        _MESH_CACHE.append(plsc.VectorSubcoreMesh(
            core_axis_name="c", subcore_axis_name="s",
            num_cores=_NC, num_subcores=_NS))
    return _MESH_CACHE[0]


def _sc_gather_count(x, src2d, dst2d, ones_blk, zeros_blk):
    @functools.partial(
        pl.kernel,
        out_type=[
            jax.ShapeDtypeStruct((_E, _F), jnp.float32),
            jax.ShapeDtypeStruct((_NP, _F), jnp.float32),
        ],
        mesh=_mesh(),
        compiler_params=_SC_PARAMS,
        scratch_types=[
            pltpu.VMEM((_GCH, _CH), jnp.int32),
            pltpu.VMEM((_SCH, _CH), jnp.int32),
            pltpu.VMEM((_STG, _F), jnp.float32),
            pltpu.VMEM((_CH, _F), jnp.float32),
            pltpu.SemaphoreType.DMA,
            pltpu.SemaphoreType.DMA,
            pltpu.VMEM_SHARED((_NP, _F), jnp.float32),
        ],
    )
    def k(x_hbm, src_hbm, dst_hbm, ones_hbm, zeros_hbm, xs_out, cnt_out,
          src_v, dst_v, stg_v, ones_v, gsem, ssem, acc):
        c = lax.axis_index("c")
        s = lax.axis_index("s")
        wid = c * _NS + s
        base = wid * _EPW
        pltpu.sync_copy(src_hbm.at[pl.ds(wid * _GCH, _GCH)], src_v)
        pltpu.sync_copy(dst_hbm.at[pl.ds(s * _SCH, _SCH)], dst_v)
        pltpu.sync_copy(ones_hbm, ones_v)
        pltpu.sync_copy(zeros_hbm, acc.at[pl.ds(s * _RPT, _RPT)])
        plsc.subcore_barrier()
        sds = []
        for j in range(_SCH):
            sds.append(pltpu.async_copy(
                ones_v, acc.at[dst_v.at[j]], ssem, add=True))
        for r in range(_GRND):
            gds = []
            for j in range(_CPR):
                gds.append(pltpu.async_copy(
                    x_hbm.at[src_v.at[r * _CPR + j]],
                    stg_v.at[pl.ds(j * _CH, _CH)], gsem))
            for d in gds:
                d.wait()
            pltpu.sync_copy(stg_v, xs_out.at[pl.ds(base + r * _STG, _STG)])
        for d in sds:
            d.wait()
        plsc.subcore_barrier()
        @pl.when(c == 0)
        def _():
            pltpu.sync_copy(acc.at[pl.ds(s * _RPT, _RPT)],
                            cnt_out.at[pl.ds(s * _RPT, _RPT)])

    return k(x, src2d, dst2d, ones_blk, zeros_blk)


def _sc_scatter_finish_gather(msg, dst2d, src2d, cnt, xroot, gvec, bvec,
                              zeros_blk, write_node_out):
    out_type = [jax.ShapeDtypeStruct((_E, _F), jnp.float32)]
    if write_node_out:
        out_type.append(jax.ShapeDtypeStruct((_NP, _F), jnp.float32))

    @functools.partial(
        pl.kernel,
        out_type=out_type,
        mesh=_mesh(),
        compiler_params=_SC_PARAMS,
        scratch_types=[
            pltpu.VMEM((_SCH, _CH), jnp.int32),
            pltpu.VMEM((_GCH, _CH), jnp.int32),
            pltpu.VMEM((_STG, _F), jnp.float32),
            pltpu.VMEM((_RPT, _F), jnp.float32),
            pltpu.VMEM((_RPT, _F), jnp.float32),
            pltpu.VMEM((_RPT, _F), jnp.float32),
            pltpu.VMEM((_F,), jnp.float32),
            pltpu.VMEM((_F,), jnp.float32),
            pltpu.SemaphoreType.DMA,
            pltpu.SemaphoreType.DMA,
            pltpu.VMEM_SHARED((_NP, _F), jnp.float32),
        ],
    )
    def k(msg_hbm, dst_hbm, src_hbm, cnt_hbm, xr_hbm, g_hbm, b_hbm,
          zeros_hbm, hs_out, *rest):
        if write_node_out:
            h_out = rest[0]
            (dst_v, src_v, stg_v, cnt_v, xr_v, acc_v,
             g_v, b_v, ssem, gsem, acc) = rest[1:]
        else:
            (dst_v, src_v, stg_v, cnt_v, xr_v, acc_v,
             g_v, b_v, ssem, gsem, acc) = rest
        c = lax.axis_index("c")
        s = lax.axis_index("s")
        wid = c * _NS + s
        base = wid * _EPW
        pltpu.sync_copy(dst_hbm.at[pl.ds(s * _SCH, _SCH)], dst_v)
        pltpu.sync_copy(src_hbm.at[pl.ds(wid * _GCH, _GCH)], src_v)
        pltpu.sync_copy(cnt_hbm.at[pl.ds(s * _RPT, _RPT)], cnt_v)
        pltpu.sync_copy(xr_hbm.at[pl.ds(s * _RPT, _RPT)], xr_v)
        pltpu.sync_copy(g_hbm, g_v)
        pltpu.sync_copy(b_hbm, b_v)
        pltpu.sync_copy(zeros_hbm, acc.at[pl.ds(s * _RPT, _RPT)])
        plsc.subcore_barrier()
        for r in range(_SRND):
            pltpu.sync_copy(
                msg_hbm.at[pl.ds(s * _EPT + r * _STG, _STG)], stg_v)
            sds = []
            for j in range(_CPR):
                sds.append(pltpu.async_copy(
                    stg_v.at[pl.ds(j * _CH, _CH)],
                    acc.at[dst_v.at[r * _CPR + j]], ssem, add=True))
            for d in sds:
                d.wait()
        plsc.subcore_barrier()
        pltpu.sync_copy(acc.at[pl.ds(s * _RPT, _RPT)], acc_v)
        gv = g_v[...]
        bv = b_v[...]

        def row(i, _):
            agg = acc_v[i] / jnp.maximum(cnt_v[i], 1.0)
            acc_v[i] = jnp.maximum((agg + xr_v[i]) * gv + bv, 0.0)
            return 0

        lax.fori_loop(0, _RPT, row, 0)
        pltpu.sync_copy(acc_v, acc.at[pl.ds(s * _RPT, _RPT)])
        if write_node_out:
            @pl.when(c == 0)
            def _():
                pltpu.sync_copy(acc_v, h_out.at[pl.ds(s * _RPT, _RPT)])
        plsc.subcore_barrier()
        for r in range(_GRND):
            gds = []
            for j in range(_CPR):
                gds.append(pltpu.async_copy(
                    acc.at[src_v.at[r * _CPR + j]],
                    stg_v.at[pl.ds(j * _CH, _CH)], gsem))
            for d in gds:
                d.wait()
            pltpu.sync_copy(stg_v, hs_out.at[pl.ds(base + r * _STG, _STG)])

    return k(msg, dst2d, src2d, cnt, xroot, gvec, bvec, zeros_blk)


def _msg_consts(w2, b2):
    f = _F
    k_idx = jnp.tile(jnp.arange(f), (f,))
    o_idx = jnp.repeat(jnp.arange(f), f)
    s_mat = (jnp.arange(f)[:, None] == k_idx[None, :]).astype(jnp.float32)
    c_mat = (o_idx[:, None] == jnp.arange(f)[None, :]).astype(jnp.float32)
    u_mat = w2.reshape(f, f, f).transpose(0, 2, 1)[:, k_idx, o_idx]
    b2m = b2.reshape(f, f)
    eye8 = jnp.eye(8, dtype=jnp.float32)
    return (jnp.kron(eye8, s_mat), jnp.kron(eye8, u_mat),
            jnp.kron(eye8, c_mat), jnp.kron(eye8, b2m))


def _bd8(w):
    return jnp.kron(jnp.eye(8, dtype=jnp.float32), w)


def _tile8(b):
    return jnp.tile(b.reshape(1, -1), (1, 8))


def _tc_msg(ea_pk, xs_pk, xn_pk, w1t8, b1t, s8, u8, c8, b8, root8, biast):
    def body(ea_ref, xs_ref, xn_ref, w1t_ref, b1_ref, s_ref, u_ref, c_ref,
             b8_ref, root_ref, biast_ref, out_ref, xr_ref):
        he = jnp.maximum(
            jnp.dot(ea_ref[...], w1t_ref[...],
                    preferred_element_type=jnp.float32) + b1_ref[...], 0.0)
        xsb = xs_ref[...]
        acc = jnp.dot(xsb, b8_ref[...], preferred_element_type=jnp.float32)
        for j in range(8):
            thj = jnp.dot(he, s_ref[:, j * 256:(j + 1) * 256],
                          preferred_element_type=jnp.float32)
            gj = jnp.dot(xsb, u_ref[:, j * 256:(j + 1) * 256],
                         preferred_element_type=jnp.float32)
            acc = acc + jnp.dot(thj * gj, c_ref[j * 256:(j + 1) * 256, :],
                                preferred_element_type=jnp.float32)
        out_ref[...] = acc
        xr_ref[...] = jnp.dot(
            xn_ref[...], root_ref[...],
            preferred_element_type=jnp.float32) + biast_ref[...]

    l = 128
    ll = 2048
    return pl.pallas_call(
        body,
        grid=(_GRID,),
        in_specs=[
            pl.BlockSpec((_BER, l), lambda i: (i, 0)),
            pl.BlockSpec((_BER, l), lambda i: (i, 0)),
            pl.BlockSpec((_NBR, l), lambda i: (i, 0)),
            pl.BlockSpec((l, l), lambda i: (0, 0)),
            pl.BlockSpec((1, l), lambda i: (0, 0)),
            pl.BlockSpec((l, ll), lambda i: (0, 0)),
            pl.BlockSpec((l, ll), lambda i: (0, 0)),
            pl.BlockSpec((ll, l), lambda i: (0, 0)),
            pl.BlockSpec((l, l), lambda i: (0, 0)),
            pl.BlockSpec((l, l), lambda i: (0, 0)),
            pl.BlockSpec((1, l), lambda i: (0, 0)),
        ],
        out_specs=[
            pl.BlockSpec((_BER, l), lambda i: (i, 0)),
            pl.BlockSpec((_NBR, l), lambda i: (i, 0)),
        ],
        out_shape=[
            jax.ShapeDtypeStruct((_EROWS, l), jnp.float32),
            jax.ShapeDtypeStruct((_NROWS, l), jnp.float32),
        ],
    )(ea_pk, xs_pk, xn_pk, w1t8, b1t, s8, u8, c8, b8, root8, biast)


def _tc_edge_mlp(hs2_pk, l1t8, l1bt, l2t8, l2bt):
    def body(h_ref, l1t_ref, l1b_ref, l2t_ref, l2b_ref, out_ref):
        h3 = jnp.maximum(
            jnp.dot(h_ref[...], l1t_ref[...],
                    preferred_element_type=jnp.float32) + l1b_ref[...], 0.0)
        out_ref[...] = jnp.dot(
            h3, l2t_ref[...], preferred_element_type=jnp.float32) + l2b_ref[...]

    l = 128
    lo = 64
    return pl.pallas_call(
        body,
        grid=(_GRID,),
        in_specs=[
            pl.BlockSpec((_BER, l), lambda i: (i, 0)),
            pl.BlockSpec((l, lo), lambda i: (0, 0)),
            pl.BlockSpec((1, lo), lambda i: (0, 0)),
            pl.BlockSpec((lo, lo), lambda i: (0, 0)),
            pl.BlockSpec((1, lo), lambda i: (0, 0)),
        ],
        out_specs=pl.BlockSpec((_BER, lo), lambda i: (i, 0)),
        out_shape=jax.ShapeDtypeStruct((_EROWS, lo), jnp.float32),
    )(hs2_pk, l1t8, l1bt, l2t8, l2bt)


def _bd16(w):
    return jnp.kron(jnp.eye(16, dtype=jnp.float32), w)


def _tile16(b):
    return jnp.tile(b.reshape(1, -1), (1, 16))


def kernel(x, edge_index, edge_attr, nn1_w1, nn1_b1, nn1_w2, nn1_b2, root1,
           bias1, bn1_g, bn1_b, nn2_w1, nn2_b1, nn2_w2, nn2_b2, root2, bias2,
           bn2_g, bn2_b, lin1_w, lin1_b, lin2_w, lin2_b):
    src2d = edge_index[0].reshape(_IDXROWS, _CH)
    dst2d = edge_index[1].reshape(_IDXROWS, _CH)
    ones_blk = jnp.ones((_CH, _F), jnp.float32)
    zeros_blk = jnp.zeros((_RPT, _F), jnp.float32)

    bn_scale = 1.0 / math.sqrt(1.0 + 1e-5)
    g1 = bn1_g * bn_scale
    g2 = bn2_g * bn_scale

    s1, u1, c1, b1m = _msg_consts(nn1_w2, nn1_b2)
    s2, u2, c2, b2m = _msg_consts(nn2_w2, nn2_b2)

    xp = jnp.pad(x, ((0, _NP - _N), (0, 0)))
    ea_pk = edge_attr.reshape(_EROWS, 128)

    xs, cnt = _sc_gather_count(xp, src2d, dst2d, ones_blk, zeros_blk)
    msg1_pk, xr1_pk = _tc_msg(ea_pk, xs.reshape(_EROWS, 128),
                              xp.reshape(_NROWS, 128),
                              _bd8(nn1_w1.T), _tile8(nn1_b1), s1, u1, c1, b1m,
                              _bd8(root1), _tile8(bias1))
    hs, h1 = _sc_scatter_finish_gather(
        msg1_pk.reshape(_E, _F), dst2d, src2d, cnt,
        xr1_pk.reshape(_NP, _F), g1, bn1_b, zeros_blk, True)
    msg2_pk, xr2_pk = _tc_msg(ea_pk, hs.reshape(_EROWS, 128),
                              h1.reshape(_NROWS, 128),
                              _bd8(nn2_w1.T), _tile8(nn2_b1), s2, u2, c2, b2m,
                              _bd8(root2), _tile8(bias2))
    hs2 = _sc_scatter_finish_gather(
        msg2_pk.reshape(_E, _F), dst2d, src2d, cnt,
        xr2_pk.reshape(_NP, _F), g2, bn2_b, zeros_blk, False)[0]
    out_pk = _tc_edge_mlp(hs2.reshape(_EROWS, 128),
                          _bd8(lin1_w.T), _tile8(lin1_b),
                          _bd8(lin2_w.T), _tile8(lin2_b))
    return out_pk.reshape(_E, _OUT)
